# baseline (speedup 1.0000x reference)
import jax
import jax.numpy as jnp
import numpy as np
from jax.experimental import pallas as pl
from jax.experimental.pallas import tpu as pltpu

_B, _C, _H, _W = 8, 3, 512, 512
_P = 16
_EMBED = 1024
_HEAD_DIM = 64
_HP = _H // _P
_WP = _W // _P
_N = _B * _HP * _WP
_K = _C * _P * _P
_M_BLK = _HP * _WP


def _fused_body(img_ref, w_ref, b_ref, o_ref):
    _Q = 8
    hq = _HP // _Q
    for q in range(_Q):
        a = img_ref[0, :, q * hq:(q + 1) * hq].astype(jnp.bfloat16)
        a = a.reshape(_C, hq, _P, _WP, _P)
        a = a.transpose(1, 3, 0, 2, 4)
        a = a.reshape(hq * _WP, _K)
        acc = jax.lax.dot_general(
            a, w_ref[...],
            (((1,), (1,)), ((), ())),
            preferred_element_type=jnp.float32,
        )
        o_ref[q * hq * _WP:(q + 1) * hq * _WP] = acc + b_ref[...]


def _project(images, w_bf16, bias_row):
    img5 = images.reshape(_B, _C, _HP, _P, _W)
    return pl.pallas_call(
        _fused_body,
        grid=(_B,),
        in_specs=[
            pl.BlockSpec((1, _C, _HP, _P, _W), lambda m: (m, 0, 0, 0, 0)),
            pl.BlockSpec((_EMBED, _K), lambda m: (0, 0)),
            pl.BlockSpec((1, _EMBED), lambda m: (0, 0)),
        ],
        out_specs=pl.BlockSpec((_M_BLK, _EMBED), lambda m: (m, 0)),
        out_shape=jax.ShapeDtypeStruct((_N, _EMBED), jnp.float32),
    )(img5, w_bf16, bias_row)


def _side_outputs():
    ys, xs = jnp.meshgrid(jnp.arange(_HP), jnp.arange(_WP), indexing="ij")
    coords = jnp.stack([ys, xs], axis=-1).reshape(-1, 2)
    patch_coords = jnp.tile(coords, (_B, 1))
    d_axis = _HEAD_DIM // 2
    n_freq = d_axis // 2
    inv_freq = 1.0 / (10000.0 ** (jnp.arange(n_freq, dtype=jnp.float32) / n_freq))
    cf = patch_coords.astype(jnp.float32)
    ang_y = cf[:, 0:1] * inv_freq[None, :]
    ang_x = cf[:, 1:2] * inv_freq[None, :]
    ang = jnp.concatenate([ang_y, ang_x], axis=-1)
    emb = jnp.concatenate([ang, ang], axis=-1)
    rope_cos, rope_sin = jnp.cos(emb), jnp.sin(emb)
    cu_seqlens = jnp.arange(_B + 1, dtype=jnp.int32) * (_HP * _WP)
    is_patch = jnp.ones((_N,), dtype=jnp.bool_)
    return cu_seqlens, patch_coords, rope_cos, rope_sin, is_patch


def kernel(images, W, b):
    w_bf = W.astype(jnp.bfloat16)
    tokens = _project(images, w_bf, b.reshape(1, _EMBED))
    cu_seqlens, patch_coords, rope_cos, rope_sin, is_patch = _side_outputs()
    return tokens, cu_seqlens, patch_coords, rope_cos, rope_sin, is_patch


# speedup vs baseline: 1.0329x; 1.0329x over previous
import jax
import jax.numpy as jnp
from jax.experimental import pallas as pl
from jax.experimental.pallas import tpu as pltpu

_B, _C, _H, _W = 8, 3, 512, 512
_P = 16
_EMBED = 1024
_HEAD_DIM = 64
_HP = _H // _P
_WP = _W // _P
_N = _B * _HP * _WP
_K = _C * _P * _P
_M_BLK = _HP * _WP


def _fused_body(img_ref, w_ref, b_ref, o_ref):
    _Q = 4
    hq = _HP // _Q
    for q in range(_Q):
        a = img_ref[0, :, q * hq:(q + 1) * hq].astype(jnp.bfloat16)
        a = a.reshape(_C, hq, _P, _WP, _P)
        a = a.transpose(1, 3, 0, 2, 4)
        a = a.reshape(hq * _WP, _K)
        acc = jax.lax.dot_general(
            a, w_ref[...],
            (((1,), (1,)), ((), ())),
            preferred_element_type=jnp.float32,
        )
        o_ref[q * hq * _WP:(q + 1) * hq * _WP] = acc + b_ref[...]


def _project(images, w_bf16, bias_row):
    img5 = images.reshape(_B, _C, _HP, _P, _W)
    return pl.pallas_call(
        _fused_body,
        grid=(_B,),
        in_specs=[
            pl.BlockSpec((1, _C, _HP, _P, _W), lambda m: (m, 0, 0, 0, 0)),
            pl.BlockSpec((_EMBED, _K), lambda m: (0, 0)),
            pl.BlockSpec((1, _EMBED), lambda m: (0, 0)),
        ],
        out_specs=pl.BlockSpec((_M_BLK, _EMBED), lambda m: (m, 0)),
        out_shape=jax.ShapeDtypeStruct((_N, _EMBED), jnp.float32),
    )(img5, w_bf16, bias_row)


def _side_outputs():
    ys, xs = jnp.meshgrid(jnp.arange(_HP), jnp.arange(_WP), indexing="ij")
    coords = jnp.stack([ys, xs], axis=-1).reshape(-1, 2)
    patch_coords = jnp.tile(coords, (_B, 1))
    d_axis = _HEAD_DIM // 2
    n_freq = d_axis // 2
    inv_freq = 1.0 / (10000.0 ** (jnp.arange(n_freq, dtype=jnp.float32) / n_freq))
    cf = patch_coords.astype(jnp.float32)
    ang_y = cf[:, 0:1] * inv_freq[None, :]
    ang_x = cf[:, 1:2] * inv_freq[None, :]
    ang = jnp.concatenate([ang_y, ang_x], axis=-1)
    emb = jnp.concatenate([ang, ang], axis=-1)
    rope_cos, rope_sin = jnp.cos(emb), jnp.sin(emb)
    cu_seqlens = jnp.arange(_B + 1, dtype=jnp.int32) * (_HP * _WP)
    is_patch = jnp.ones((_N,), dtype=jnp.bool_)
    return cu_seqlens, patch_coords, rope_cos, rope_sin, is_patch


def kernel(images, W, b):
    w_bf = W.astype(jnp.bfloat16)
    tokens = _project(images, w_bf, b.reshape(1, _EMBED))
    cu_seqlens, patch_coords, rope_cos, rope_sin, is_patch = _side_outputs()
    return tokens, cu_seqlens, patch_coords, rope_cos, rope_sin, is_patch
